# Initial kernel scaffold; baseline (speedup 1.0000x reference)
import jax
import jax.numpy as jnp
from jax.experimental import pallas as pl


def kernel(edge_attr, tables):
    raise NotImplementedError("write your pallas kernel here")


# speedup vs baseline: 8.3908x; 8.3908x over previous
import functools

import jax
import jax.numpy as jnp
from jax import lax
from jax.experimental import pallas as pl
from jax.experimental.pallas import tpu as pltpu
from jax.experimental.pallas import tpu_sc as plsc

NUM_TABLES = 9
VOCAB = 100
HIDDEN = 64
LANES = 16
NUM_COL_BLOCKS = HIDDEN // LANES


def _sc_body(num_workers, edges_per_worker, chunk, num_chunks,
             attr_hbm, table_hbm, out_hbm, table_v, attr_v, out_v):
  num_cores = num_workers // 16
  wid = lax.axis_index("s") * num_cores + lax.axis_index("c")

  pltpu.sync_copy(table_hbm, table_v)

  cols = [lax.iota(jnp.int32, 16) + 16 * b for b in range(NUM_COL_BLOCKS)]
  table_off = lax.iota(jnp.int32, LANES) * (VOCAB * HIDDEN)

  def chunk_body(g, carry):
    base = wid * edges_per_worker + g * chunk
    pltpu.sync_copy(attr_hbm.at[pl.ds(base * NUM_TABLES, chunk * NUM_TABLES)],
                    attr_v.at[pl.ds(0, chunk * NUM_TABLES)])

    def edge_body(e, carry2):
      av = attr_v[pl.ds(e * NUM_TABLES, LANES)]
      rows = av * HIDDEN + table_off
      accs = [None] * NUM_COL_BLOCKS
      for i in range(NUM_TABLES):
        rv = jnp.full((LANES,), rows[i], jnp.int32)
        for b in range(NUM_COL_BLOCKS):
          v = plsc.load_gather(table_v, [rv + cols[b]])
          accs[b] = v if i == 0 else accs[b] + v
      for b in range(NUM_COL_BLOCKS):
        out_v[pl.ds(e * HIDDEN + 16 * b, LANES)] = accs[b]
      return carry2

    lax.fori_loop(0, chunk, edge_body, 0)
    pltpu.sync_copy(out_v, out_hbm.at[pl.ds(base * HIDDEN, chunk * HIDDEN)])
    return carry

  lax.fori_loop(0, num_chunks, chunk_body, 0)


@jax.jit
def kernel(edge_attr, tables):
  e_total = edge_attr.shape[0]
  info = plsc.get_sparse_core_info()
  num_workers = info.num_cores * info.num_subcores
  assert e_total % num_workers == 0
  edges_per_worker = e_total // num_workers
  chunk = 200
  assert edges_per_worker % chunk == 0
  num_chunks = edges_per_worker // chunk

  attr = edge_attr.astype(jnp.int32).reshape(-1)
  tab = tables.astype(jnp.float32).reshape(-1)

  mesh = plsc.VectorSubcoreMesh(core_axis_name="c", subcore_axis_name="s")
  call = pl.kernel(
      functools.partial(_sc_body, num_workers, edges_per_worker, chunk,
                        num_chunks),
      out_type=jax.ShapeDtypeStruct((e_total * HIDDEN,), jnp.float32),
      mesh=mesh,
      compiler_params=pltpu.CompilerParams(needs_layout_passes=False),
      scratch_types=[
          pltpu.VMEM((NUM_TABLES * VOCAB * HIDDEN,), jnp.float32),
          pltpu.VMEM((chunk * NUM_TABLES + LANES,), jnp.int32),
          pltpu.VMEM((chunk * HIDDEN,), jnp.float32),
      ],
  )
  out = call(attr, tab)
  return out.reshape(e_total, HIDDEN)


# speedup vs baseline: 11.8308x; 1.4100x over previous
import functools

import jax
import jax.numpy as jnp
from jax import lax
from jax.experimental import pallas as pl
from jax.experimental.pallas import tpu as pltpu
from jax.experimental.pallas import tpu_sc as plsc

NUM_TABLES = 9
VOCAB = 100
HIDDEN = 64
LANES = 16


def _sc_body(num_workers, edges_per_worker, chunk, num_chunks,
             attr_hbm, table_hbm, out_hbm, table_v, attr0, attr1, out0, out1,
             sem_a0, sem_a1, sem_o0, sem_o1):
  num_cores = num_workers // 16
  wid = lax.axis_index("s") * num_cores + lax.axis_index("c")

  pltpu.sync_copy(table_hbm, table_v)

  table_off = lax.iota(jnp.int32, LANES) * (VOCAB * HIDDEN // 2)

  def attr_copy(g, buf, sem):
    base = wid * edges_per_worker + g * chunk
    return pltpu.make_async_copy(
        attr_hbm.at[pl.ds(base * NUM_TABLES, chunk * NUM_TABLES)],
        buf.at[pl.ds(0, chunk * NUM_TABLES)], sem)

  def out_copy(g, buf, sem):
    base = wid * edges_per_worker + g * chunk
    return pltpu.make_async_copy(
        buf, out_hbm.at[pl.ds(base * HIDDEN, chunk * HIDDEN)], sem)

  cols = [lax.iota(jnp.int32, LANES) + LANES * b for b in range(2)]

  def compute(attr_v, out_v):
    @plsc.parallel_loop(0, chunk, 1, unroll=2)
    def edge_body(e):
      av = attr_v[pl.ds(e * NUM_TABLES, LANES)]
      rows = av * (HIDDEN // 2) + table_off
      accs = [None] * 4
      for i in range(NUM_TABLES):
        rv = jnp.full((LANES,), rows[i], jnp.int32)
        for h in range(2):
          iw = plsc.load_gather(table_v, [rv + cols[h]])
          lo = plsc.bitcast(iw << 16, jnp.float32)
          hi = plsc.bitcast(iw & jnp.int32(-65536), jnp.float32)
          if i == 0:
            accs[2 * h] = lo
            accs[2 * h + 1] = hi
          else:
            accs[2 * h] = accs[2 * h] + lo
            accs[2 * h + 1] = accs[2 * h + 1] + hi
      for b in range(4):
        out_v[pl.ds(e * HIDDEN + 16 * b, LANES)] = accs[b]

  num_pairs = num_chunks // 2
  attr_copy(0, attr0, sem_a0).start()

  def pair_body(gg, carry):
    g0 = gg * 2
    g1 = g0 + 1

    attr_copy(g0, attr0, sem_a0).wait()
    attr_copy(g1, attr1, sem_a1).start()

    @pl.when(gg > 0)
    def _():
      out_copy(g0 - 2, out0, sem_o0).wait()

    compute(attr0, out0)
    out_copy(g0, out0, sem_o0).start()

    attr_copy(g1, attr1, sem_a1).wait()
    attr_copy(g1 + 1, attr0, sem_a0).start()

    @pl.when(gg > 0)
    def _():
      out_copy(g1 - 2, out1, sem_o1).wait()

    compute(attr1, out1)
    out_copy(g1, out1, sem_o1).start()
    return carry

  lax.fori_loop(0, num_pairs, pair_body, 0)

  gt = num_chunks - 1
  attr_copy(gt, attr0, sem_a0).wait()
  out_copy(gt - 2, out0, sem_o0).wait()
  compute(attr0, out0)
  out_copy(gt, out0, sem_o0).start()
  out_copy(gt, out0, sem_o0).wait()
  out_copy(gt - 1, out1, sem_o1).wait()


@jax.jit
def kernel(edge_attr, tables):
  e_total = edge_attr.shape[0]
  info = plsc.get_sparse_core_info()
  num_workers = info.num_cores * info.num_subcores
  assert e_total % num_workers == 0
  edges_per_worker = e_total // num_workers
  chunk = 200
  assert edges_per_worker % chunk == 0
  num_chunks = edges_per_worker // chunk
  assert num_chunks % 2 == 1 and num_chunks > 2

  attr = edge_attr.astype(jnp.int32).reshape(-1)
  tab = tables.astype(jnp.float32).reshape(NUM_TABLES * VOCAB, 2, 2, LANES)
  tab = tab.transpose(0, 1, 3, 2).reshape(-1).astype(jnp.bfloat16)
  tab = jax.lax.bitcast_convert_type(tab.reshape(-1, 2), jnp.int32)

  mesh = plsc.VectorSubcoreMesh(core_axis_name="c", subcore_axis_name="s")
  call = pl.kernel(
      functools.partial(_sc_body, num_workers, edges_per_worker, chunk,
                        num_chunks),
      out_type=jax.ShapeDtypeStruct((e_total * HIDDEN,), jnp.float32),
      mesh=mesh,
      compiler_params=pltpu.CompilerParams(needs_layout_passes=False),
      scratch_types=[
          pltpu.VMEM((NUM_TABLES * VOCAB * HIDDEN // 2,), jnp.int32),
          pltpu.VMEM((chunk * NUM_TABLES + LANES,), jnp.int32),
          pltpu.VMEM((chunk * NUM_TABLES + LANES,), jnp.int32),
          pltpu.VMEM((chunk * HIDDEN,), jnp.float32),
          pltpu.VMEM((chunk * HIDDEN,), jnp.float32),
          pltpu.SemaphoreType.DMA,
          pltpu.SemaphoreType.DMA,
          pltpu.SemaphoreType.DMA,
          pltpu.SemaphoreType.DMA,
      ],
  )
  out = call(attr, tab)
  return out.reshape(e_total, HIDDEN)
